# baseline (speedup 1.0000x reference)
import functools

import numpy as np
import jax
import jax.numpy as jnp
from jax import lax
from jax.experimental import pallas as pl
from jax.experimental.pallas import tpu as pltpu
from jax.experimental.pallas import tpu_sc as plsc

EMB = 64
HID = 256
OUT = 48
B = 16384
WIN = 5

NW = 32
CHUNK = 128
NSPLIT = 4
BS = B // NSPLIT
ROWS_PER_W = BS * 6 // NW
NCHUNK = ROWS_PER_W // CHUNK
B_PER_W = BS // NW
OROWS = BS * 3


def _pattern() -> np.ndarray:
    c = np.arange(ROWS_PER_W) // 128
    p = np.arange(ROWS_PER_W) % 128
    k = 128 * c + 2 * (p % 64) + p // 64
    rb = k // 48
    rem = k % 48
    t = rem // 16
    r = (rem % 16) // 2
    h = rem % 2
    w = np.minimum(2 * t + h, 4)
    return np.stack([8 * rb + r, w]).astype(np.int32)


_PATTERN = _pattern()


NBUF = 4


@functools.cache
def _build_sc_gather():
    mesh = plsc.VectorSubcoreMesh(core_axis_name="c", subcore_axis_name="s")

    @functools.partial(
        pl.kernel,
        out_type=jax.ShapeDtypeStruct((OROWS, 128), jnp.float32),
        mesh=mesh,
        scratch_types=[
            pltpu.VMEM((B_PER_W, WIN), jnp.int32),
            pltpu.VMEM((2, ROWS_PER_W), jnp.int32),
            pltpu.VMEM((ROWS_PER_W,), jnp.int32),
            pltpu.VMEM((NBUF, CHUNK, EMB), jnp.float32),
            pltpu.SemaphoreType.DMA,
        ] + [pltpu.SemaphoreType.DMA] * NBUF
        + [pltpu.SemaphoreType.DMA] * NBUF,
        compiler_params=pltpu.CompilerParams(
            use_tc_tiling_on_sc=False, needs_layout_passes=False
        ),
    )
    def _sc_gather(x_hbm, table_hbm, pat_hbm, out_hbm, x_v, pat_v, idx_v,
                   rows_v, *sems):
        gsems = sems[:NBUF]
        osems = sems[NBUF:]
        wid = lax.axis_index("s") * 2 + lax.axis_index("c")
        pltpu.sync_copy(x_hbm.at[pl.ds(wid * B_PER_W, B_PER_W)], x_v)
        pltpu.sync_copy(pat_hbm, pat_v)
        for i in range(ROWS_PER_W // 16):
            rsel = pat_v[0, pl.ds(16 * i, 16)]
            csel = pat_v[1, pl.ds(16 * i, 16)]
            vals = plsc.load_gather(x_v, [rsel, csel])
            idx_v[pl.ds(16 * i, 16)] = vals
        obase = wid * (OROWS // NW)

        def fire(j):
            pltpu.async_copy(
                table_hbm.at[idx_v.at[pl.ds(j * CHUNK, CHUNK)]],
                rows_v.at[j % NBUF], gsems[j % NBUF],
            )

        def wait_writes(j):
            slot = j % NBUF
            dst = out_hbm.at[pl.ds(obase + j * 64, 64)]
            pltpu.make_async_copy(
                rows_v.at[slot, pl.ds(0, 64)], dst.at[:, pl.ds(0, 64)],
                osems[slot],
            ).wait()
            pltpu.make_async_copy(
                rows_v.at[slot, pl.ds(64, 64)], dst.at[:, pl.ds(64, 64)],
                osems[slot],
            ).wait()

        for j in range(min(NBUF, NCHUNK)):
            fire(j)
        for j in range(NCHUNK):
            slot = j % NBUF
            buf = rows_v.at[slot]
            pltpu.make_async_copy(
                table_hbm.at[idx_v.at[pl.ds(j * CHUNK, CHUNK)]], buf,
                gsems[slot],
            ).wait()
            dst = out_hbm.at[pl.ds(obase + j * 64, 64)]
            pltpu.async_copy(buf.at[pl.ds(0, 64)], dst.at[:, pl.ds(0, 64)],
                             osems[slot])
            pltpu.async_copy(buf.at[pl.ds(64, 64)], dst.at[:, pl.ds(64, 64)],
                             osems[slot])
            nxt = j + 1
            if nxt < NCHUNK and nxt >= NBUF:
                wait_writes(nxt - NBUF)
                fire(nxt)
        for j in range(max(0, NCHUNK - NBUF), NCHUNK):
            wait_writes(j)

    return _sc_gather


BLKR = 256


def _mlp_body(flat_ref, w1_ref, b1_ref, w2_ref, b2_ref, out_ref):
    x4 = flat_ref[...].reshape(BLKR, 3, 8, 128)
    nrow = BLKR * 8
    acc = b1_ref[...]
    acc = acc + jnp.dot(x4[:, 0].reshape(nrow, 128), w1_ref[pl.ds(0, 128)],
                        preferred_element_type=jnp.float32)
    acc = acc + jnp.dot(x4[:, 1].reshape(nrow, 128), w1_ref[pl.ds(128, 128)],
                        preferred_element_type=jnp.float32)
    acc = acc + jnp.dot(x4[:, 2].reshape(nrow, 128)[:, :64],
                        w1_ref[pl.ds(256, 64)],
                        preferred_element_type=jnp.float32)
    h = jnp.tanh(acc)
    out_ref[...] = (
        jnp.dot(h, w2_ref[...], preferred_element_type=jnp.float32) + b2_ref[...]
    )


def _mlp(flat2d, W1, b1, W2, b2):
    return pl.pallas_call(
        _mlp_body,
        grid=(BS // (8 * BLKR),),
        in_specs=[
            pl.BlockSpec((3 * 8 * BLKR, 128), lambda i: (i, 0)),
            pl.BlockSpec((WIN * EMB, HID), lambda i: (0, 0)),
            pl.BlockSpec((1, HID), lambda i: (0, 0)),
            pl.BlockSpec((HID, OUT), lambda i: (0, 0)),
            pl.BlockSpec((1, OUT), lambda i: (0, 0)),
        ],
        out_specs=pl.BlockSpec((8 * BLKR, OUT), lambda i: (i, 0)),
        out_shape=jax.ShapeDtypeStruct((BS, OUT), jnp.float32),
    )(flat2d, W1, b1, W2, b2)


@jax.jit
def kernel(x, table, W1, b1, W2, b2):
    pat = jnp.asarray(_PATTERN)
    b1r = b1.reshape(1, HID)
    b2r = b2.reshape(1, OUT)
    outs = []
    for s in range(NSPLIT):
        flat2d = _build_sc_gather()(x[s * BS:(s + 1) * BS], table, pat)
        outs.append(_mlp(flat2d, W1, b1r, W2, b2r))
    return jnp.concatenate(outs, axis=0)


# speedup vs baseline: 1.0415x; 1.0415x over previous
import functools

import numpy as np
import jax
import jax.numpy as jnp
from jax import lax
from jax.experimental import pallas as pl
from jax.experimental.pallas import tpu as pltpu
from jax.experimental.pallas import tpu_sc as plsc

EMB = 64
HID = 256
OUT = 48
B = 16384
WIN = 5

NW = 32
CHUNK = 128
NSPLIT = 2
BS = B // NSPLIT
ROWS_PER_W = BS * 6 // NW
NCHUNK = ROWS_PER_W // CHUNK
B_PER_W = BS // NW
OROWS = BS * 3


def _pattern() -> np.ndarray:
    c = np.arange(ROWS_PER_W) // 128
    p = np.arange(ROWS_PER_W) % 128
    k = 128 * c + 2 * (p % 64) + p // 64
    rb = k // 48
    rem = k % 48
    t = rem // 16
    r = (rem % 16) // 2
    h = rem % 2
    w = np.minimum(2 * t + h, 4)
    return np.stack([8 * rb + r, w]).astype(np.int32)


_PATTERN = _pattern()


NBUF = 6


@functools.cache
def _build_sc_gather():
    mesh = plsc.VectorSubcoreMesh(core_axis_name="c", subcore_axis_name="s")

    @functools.partial(
        pl.kernel,
        out_type=jax.ShapeDtypeStruct((OROWS, 128), jnp.float32),
        mesh=mesh,
        scratch_types=[
            pltpu.VMEM((B_PER_W, WIN), jnp.int32),
            pltpu.VMEM((2, ROWS_PER_W), jnp.int32),
            pltpu.VMEM((ROWS_PER_W,), jnp.int32),
            pltpu.VMEM((NBUF, CHUNK, EMB), jnp.float32),
            pltpu.SemaphoreType.DMA,
        ] + [pltpu.SemaphoreType.DMA] * NBUF
        + [pltpu.SemaphoreType.DMA] * NBUF,
        compiler_params=pltpu.CompilerParams(
            use_tc_tiling_on_sc=False, needs_layout_passes=False
        ),
    )
    def _sc_gather(x_hbm, table_hbm, pat_hbm, out_hbm, x_v, pat_v, idx_v,
                   rows_v, *sems):
        gsems = sems[:NBUF]
        osems = sems[NBUF:]
        wid = lax.axis_index("s") * 2 + lax.axis_index("c")
        pltpu.sync_copy(x_hbm.at[pl.ds(wid * B_PER_W, B_PER_W)], x_v)
        pltpu.sync_copy(pat_hbm, pat_v)
        for i in range(ROWS_PER_W // 16):
            rsel = pat_v[0, pl.ds(16 * i, 16)]
            csel = pat_v[1, pl.ds(16 * i, 16)]
            vals = plsc.load_gather(x_v, [rsel, csel])
            idx_v[pl.ds(16 * i, 16)] = vals
        obase = wid * (OROWS // NW)

        def fire(j):
            pltpu.async_copy(
                table_hbm.at[idx_v.at[pl.ds(j * CHUNK, CHUNK)]],
                rows_v.at[j % NBUF], gsems[j % NBUF],
            )

        def wait_writes(j):
            slot = j % NBUF
            dst = out_hbm.at[pl.ds(obase + j * 64, 64)]
            pltpu.make_async_copy(
                rows_v.at[slot, pl.ds(0, 64)], dst.at[:, pl.ds(0, 64)],
                osems[slot],
            ).wait()
            pltpu.make_async_copy(
                rows_v.at[slot, pl.ds(64, 64)], dst.at[:, pl.ds(64, 64)],
                osems[slot],
            ).wait()

        for j in range(min(NBUF, NCHUNK)):
            fire(j)
        for j in range(NCHUNK):
            slot = j % NBUF
            buf = rows_v.at[slot]
            pltpu.make_async_copy(
                table_hbm.at[idx_v.at[pl.ds(j * CHUNK, CHUNK)]], buf,
                gsems[slot],
            ).wait()
            dst = out_hbm.at[pl.ds(obase + j * 64, 64)]
            pltpu.async_copy(buf.at[pl.ds(0, 64)], dst.at[:, pl.ds(0, 64)],
                             osems[slot])
            pltpu.async_copy(buf.at[pl.ds(64, 64)], dst.at[:, pl.ds(64, 64)],
                             osems[slot])
            nxt = j + 1
            if nxt < NCHUNK and nxt >= NBUF:
                wait_writes(nxt - NBUF)
                fire(nxt)
        for j in range(max(0, NCHUNK - NBUF), NCHUNK):
            wait_writes(j)

    return _sc_gather


BLKR = 256


def _mlp_body(flat_ref, w1_ref, b1_ref, w2_ref, b2_ref, out_ref):
    x4 = flat_ref[...].reshape(BLKR, 3, 8, 128)
    nrow = BLKR * 8
    acc = b1_ref[...]
    acc = acc + jnp.dot(x4[:, 0].reshape(nrow, 128), w1_ref[pl.ds(0, 128)],
                        preferred_element_type=jnp.float32)
    acc = acc + jnp.dot(x4[:, 1].reshape(nrow, 128), w1_ref[pl.ds(128, 128)],
                        preferred_element_type=jnp.float32)
    acc = acc + jnp.dot(x4[:, 2].reshape(nrow, 128)[:, :64],
                        w1_ref[pl.ds(256, 64)],
                        preferred_element_type=jnp.float32)
    h = jnp.tanh(acc)
    out_ref[...] = (
        jnp.dot(h, w2_ref[...], preferred_element_type=jnp.float32) + b2_ref[...]
    )


def _mlp(flat2d, W1, b1, W2, b2):
    return pl.pallas_call(
        _mlp_body,
        grid=(BS // (8 * BLKR),),
        in_specs=[
            pl.BlockSpec((3 * 8 * BLKR, 128), lambda i: (i, 0)),
            pl.BlockSpec((WIN * EMB, HID), lambda i: (0, 0)),
            pl.BlockSpec((1, HID), lambda i: (0, 0)),
            pl.BlockSpec((HID, OUT), lambda i: (0, 0)),
            pl.BlockSpec((1, OUT), lambda i: (0, 0)),
        ],
        out_specs=pl.BlockSpec((8 * BLKR, OUT), lambda i: (i, 0)),
        out_shape=jax.ShapeDtypeStruct((BS, OUT), jnp.float32),
    )(flat2d, W1, b1, W2, b2)


@jax.jit
def kernel(x, table, W1, b1, W2, b2):
    pat = jnp.asarray(_PATTERN)
    b1r = b1.reshape(1, HID)
    b2r = b2.reshape(1, OUT)
    outs = []
    for s in range(NSPLIT):
        flat2d = _build_sc_gather()(x[s * BS:(s + 1) * BS], table, pat)
        outs.append(_mlp(flat2d, W1, b1r, W2, b2r))
    return jnp.concatenate(outs, axis=0)


# speedup vs baseline: 1.0554x; 1.0134x over previous
import functools

import numpy as np
import jax
import jax.numpy as jnp
from jax import lax
from jax.experimental import pallas as pl
from jax.experimental.pallas import tpu as pltpu
from jax.experimental.pallas import tpu_sc as plsc

EMB = 64
HID = 256
OUT = 48
B = 16384
WIN = 5

NW = 32
CHUNK = 128
NSPLIT = 2
BS = B // NSPLIT
ROWS_PER_W = BS * 6 // NW
NCHUNK = ROWS_PER_W // CHUNK
B_PER_W = BS // NW
OROWS = BS * 3


def _pattern() -> np.ndarray:
    c = np.arange(ROWS_PER_W) // 128
    p = np.arange(ROWS_PER_W) % 128
    k = 128 * c + 2 * (p % 64) + p // 64
    rb = k // 48
    rem = k % 48
    t = rem // 16
    r = (rem % 16) // 2
    h = rem % 2
    w = np.minimum(2 * t + h, 4)
    return np.stack([8 * rb + r, w]).astype(np.int32)


_PATTERN = _pattern()


NBUF = 8


@functools.cache
def _build_sc_gather():
    mesh = plsc.VectorSubcoreMesh(core_axis_name="c", subcore_axis_name="s")

    @functools.partial(
        pl.kernel,
        out_type=jax.ShapeDtypeStruct((OROWS, 128), jnp.float32),
        mesh=mesh,
        scratch_types=[
            pltpu.VMEM((B_PER_W, WIN), jnp.int32),
            pltpu.VMEM((2, ROWS_PER_W), jnp.int32),
            pltpu.VMEM((ROWS_PER_W,), jnp.int32),
            pltpu.VMEM((NBUF, CHUNK, EMB), jnp.float32),
            pltpu.SemaphoreType.DMA,
        ] + [pltpu.SemaphoreType.DMA] * NBUF
        + [pltpu.SemaphoreType.DMA] * NBUF,
        compiler_params=pltpu.CompilerParams(
            use_tc_tiling_on_sc=False, needs_layout_passes=False
        ),
    )
    def _sc_gather(x_hbm, table_hbm, pat_hbm, out_hbm, x_v, pat_v, idx_v,
                   rows_v, *sems):
        gsems = sems[:NBUF]
        osems = sems[NBUF:]
        wid = lax.axis_index("s") * 2 + lax.axis_index("c")
        pltpu.sync_copy(x_hbm.at[pl.ds(wid * B_PER_W, B_PER_W)], x_v)
        pltpu.sync_copy(pat_hbm, pat_v)
        for i in range(ROWS_PER_W // 16):
            rsel = pat_v[0, pl.ds(16 * i, 16)]
            csel = pat_v[1, pl.ds(16 * i, 16)]
            vals = plsc.load_gather(x_v, [rsel, csel])
            idx_v[pl.ds(16 * i, 16)] = vals
        obase = wid * (OROWS // NW)

        def fire(j):
            pltpu.async_copy(
                table_hbm.at[idx_v.at[pl.ds(j * CHUNK, CHUNK)]],
                rows_v.at[j % NBUF], gsems[j % NBUF],
            )

        def wait_writes(j):
            slot = j % NBUF
            dst = out_hbm.at[pl.ds(obase + j * 64, 64)]
            pltpu.make_async_copy(
                rows_v.at[slot, pl.ds(0, 64)], dst.at[:, pl.ds(0, 64)],
                osems[slot],
            ).wait()
            pltpu.make_async_copy(
                rows_v.at[slot, pl.ds(64, 64)], dst.at[:, pl.ds(64, 64)],
                osems[slot],
            ).wait()

        for j in range(min(NBUF, NCHUNK)):
            fire(j)
        for j in range(NCHUNK):
            slot = j % NBUF
            buf = rows_v.at[slot]
            pltpu.make_async_copy(
                table_hbm.at[idx_v.at[pl.ds(j * CHUNK, CHUNK)]], buf,
                gsems[slot],
            ).wait()
            dst = out_hbm.at[pl.ds(obase + j * 64, 64)]
            pltpu.async_copy(buf.at[pl.ds(0, 64)], dst.at[:, pl.ds(0, 64)],
                             osems[slot])
            pltpu.async_copy(buf.at[pl.ds(64, 64)], dst.at[:, pl.ds(64, 64)],
                             osems[slot])
            nxt = j + 1
            if nxt < NCHUNK and nxt >= NBUF:
                wait_writes(nxt - NBUF)
                fire(nxt)
        for j in range(max(0, NCHUNK - NBUF), NCHUNK):
            wait_writes(j)

    return _sc_gather


BLKR = 256


def _mlp_body(flat_ref, w1_ref, b1_ref, w2_ref, b2_ref, out_ref):
    x4 = flat_ref[...].reshape(BLKR, 3, 8, 128)
    nrow = BLKR * 8
    acc = b1_ref[...]
    acc = acc + jnp.dot(x4[:, 0].reshape(nrow, 128), w1_ref[pl.ds(0, 128)],
                        preferred_element_type=jnp.float32)
    acc = acc + jnp.dot(x4[:, 1].reshape(nrow, 128), w1_ref[pl.ds(128, 128)],
                        preferred_element_type=jnp.float32)
    acc = acc + jnp.dot(x4[:, 2].reshape(nrow, 128)[:, :64],
                        w1_ref[pl.ds(256, 64)],
                        preferred_element_type=jnp.float32)
    h = jnp.tanh(acc)
    out_ref[...] = (
        jnp.dot(h, w2_ref[...], preferred_element_type=jnp.float32) + b2_ref[...]
    )


def _mlp(flat2d, W1, b1, W2, b2):
    return pl.pallas_call(
        _mlp_body,
        grid=(BS // (8 * BLKR),),
        in_specs=[
            pl.BlockSpec((3 * 8 * BLKR, 128), lambda i: (i, 0)),
            pl.BlockSpec((WIN * EMB, HID), lambda i: (0, 0)),
            pl.BlockSpec((1, HID), lambda i: (0, 0)),
            pl.BlockSpec((HID, OUT), lambda i: (0, 0)),
            pl.BlockSpec((1, OUT), lambda i: (0, 0)),
        ],
        out_specs=pl.BlockSpec((8 * BLKR, OUT), lambda i: (i, 0)),
        out_shape=jax.ShapeDtypeStruct((BS, OUT), jnp.float32),
    )(flat2d, W1, b1, W2, b2)


@jax.jit
def kernel(x, table, W1, b1, W2, b2):
    pat = jnp.asarray(_PATTERN)
    b1r = b1.reshape(1, HID)
    b2r = b2.reshape(1, OUT)
    outs = []
    for s in range(NSPLIT):
        flat2d = _build_sc_gather()(x[s * BS:(s + 1) * BS], table, pat)
        outs.append(_mlp(flat2d, W1, b1r, W2, b2r))
    return jnp.concatenate(outs, axis=0)


# speedup vs baseline: 1.0856x; 1.0286x over previous
import functools

import numpy as np
import jax
import jax.numpy as jnp
from jax import lax
from jax.experimental import pallas as pl
from jax.experimental.pallas import tpu as pltpu
from jax.experimental.pallas import tpu_sc as plsc

EMB = 64
HID = 256
OUT = 48
B = 16384
WIN = 5

NW = 32
CHUNK = 128
NSPLIT = 2
BS = B // NSPLIT
ROWS_PER_W = BS * 6 // NW
NCHUNK = ROWS_PER_W // CHUNK
B_PER_W = BS // NW
OROWS = BS * 3


def _pattern() -> np.ndarray:
    c = np.arange(ROWS_PER_W) // 128
    p = np.arange(ROWS_PER_W) % 128
    k = 128 * c + 2 * (p % 64) + p // 64
    rb = k // 48
    rem = k % 48
    t = rem // 16
    r = (rem % 16) // 2
    h = rem % 2
    w = np.minimum(2 * t + h, 4)
    return np.stack([8 * rb + r, w]).astype(np.int32)


_PATTERN = _pattern()


NBUF = 12


@functools.cache
def _build_sc_gather():
    mesh = plsc.VectorSubcoreMesh(core_axis_name="c", subcore_axis_name="s")

    @functools.partial(
        pl.kernel,
        out_type=jax.ShapeDtypeStruct((OROWS, 128), jnp.float32),
        mesh=mesh,
        scratch_types=[
            pltpu.VMEM((B_PER_W, WIN), jnp.int32),
            pltpu.VMEM((2, ROWS_PER_W), jnp.int32),
            pltpu.VMEM((ROWS_PER_W,), jnp.int32),
            pltpu.VMEM((NBUF, CHUNK, EMB), jnp.float32),
            pltpu.SemaphoreType.DMA,
        ] + [pltpu.SemaphoreType.DMA] * NBUF
        + [pltpu.SemaphoreType.DMA] * NBUF,
        compiler_params=pltpu.CompilerParams(
            use_tc_tiling_on_sc=False, needs_layout_passes=False
        ),
    )
    def _sc_gather(x_hbm, table_hbm, pat_hbm, out_hbm, x_v, pat_v, idx_v,
                   rows_v, *sems):
        gsems = sems[:NBUF]
        osems = sems[NBUF:]
        wid = lax.axis_index("s") * 2 + lax.axis_index("c")
        pltpu.sync_copy(x_hbm.at[pl.ds(wid * B_PER_W, B_PER_W)], x_v)
        pltpu.sync_copy(pat_hbm, pat_v)
        for i in range(ROWS_PER_W // 16):
            rsel = pat_v[0, pl.ds(16 * i, 16)]
            csel = pat_v[1, pl.ds(16 * i, 16)]
            vals = plsc.load_gather(x_v, [rsel, csel])
            idx_v[pl.ds(16 * i, 16)] = vals
        obase = wid * (OROWS // NW)

        def fire(j):
            pltpu.async_copy(
                table_hbm.at[idx_v.at[pl.ds(j * CHUNK, CHUNK)]],
                rows_v.at[j % NBUF], gsems[j % NBUF],
            )

        def wait_writes(j):
            slot = j % NBUF
            dst = out_hbm.at[pl.ds(obase + j * 64, 64)]
            pltpu.make_async_copy(
                rows_v.at[slot, pl.ds(0, 64)], dst.at[:, pl.ds(0, 64)],
                osems[slot],
            ).wait()
            pltpu.make_async_copy(
                rows_v.at[slot, pl.ds(64, 64)], dst.at[:, pl.ds(64, 64)],
                osems[slot],
            ).wait()

        for j in range(min(NBUF, NCHUNK)):
            fire(j)
        for j in range(NCHUNK):
            slot = j % NBUF
            buf = rows_v.at[slot]
            pltpu.make_async_copy(
                table_hbm.at[idx_v.at[pl.ds(j * CHUNK, CHUNK)]], buf,
                gsems[slot],
            ).wait()
            dst = out_hbm.at[pl.ds(obase + j * 64, 64)]
            pltpu.async_copy(buf.at[pl.ds(0, 64)], dst.at[:, pl.ds(0, 64)],
                             osems[slot])
            pltpu.async_copy(buf.at[pl.ds(64, 64)], dst.at[:, pl.ds(64, 64)],
                             osems[slot])
            nxt = j + 1
            if nxt < NCHUNK and nxt >= NBUF:
                wait_writes(nxt - NBUF)
                fire(nxt)
        for j in range(max(0, NCHUNK - NBUF), NCHUNK):
            wait_writes(j)

    return _sc_gather


BLKR = 256


def _mlp_body(flat_ref, w1_ref, b1_ref, w2_ref, b2_ref, out_ref):
    x4 = flat_ref[...].reshape(BLKR, 3, 8, 128)
    nrow = BLKR * 8
    acc = b1_ref[...]
    acc = acc + jnp.dot(x4[:, 0].reshape(nrow, 128), w1_ref[pl.ds(0, 128)],
                        preferred_element_type=jnp.float32)
    acc = acc + jnp.dot(x4[:, 1].reshape(nrow, 128), w1_ref[pl.ds(128, 128)],
                        preferred_element_type=jnp.float32)
    acc = acc + jnp.dot(x4[:, 2].reshape(nrow, 128)[:, :64],
                        w1_ref[pl.ds(256, 64)],
                        preferred_element_type=jnp.float32)
    h = jnp.tanh(acc)
    out_ref[...] = (
        jnp.dot(h, w2_ref[...], preferred_element_type=jnp.float32) + b2_ref[...]
    )


def _mlp(flat2d, W1, b1, W2, b2):
    return pl.pallas_call(
        _mlp_body,
        grid=(BS // (8 * BLKR),),
        in_specs=[
            pl.BlockSpec((3 * 8 * BLKR, 128), lambda i: (i, 0)),
            pl.BlockSpec((WIN * EMB, HID), lambda i: (0, 0)),
            pl.BlockSpec((1, HID), lambda i: (0, 0)),
            pl.BlockSpec((HID, OUT), lambda i: (0, 0)),
            pl.BlockSpec((1, OUT), lambda i: (0, 0)),
        ],
        out_specs=pl.BlockSpec((8 * BLKR, OUT), lambda i: (i, 0)),
        out_shape=jax.ShapeDtypeStruct((BS, OUT), jnp.float32),
    )(flat2d, W1, b1, W2, b2)


@jax.jit
def kernel(x, table, W1, b1, W2, b2):
    pat = jnp.asarray(_PATTERN)
    b1r = b1.reshape(1, HID)
    b2r = b2.reshape(1, OUT)
    outs = []
    for s in range(NSPLIT):
        flat2d = _build_sc_gather()(x[s * BS:(s + 1) * BS], table, pat)
        outs.append(_mlp(flat2d, W1, b1r, W2, b2r))
    return jnp.concatenate(outs, axis=0)
